# baseline (speedup 1.0000x reference)
import functools

import jax
import jax.numpy as jnp
from jax import lax
from jax.experimental import pallas as pl
from jax.experimental.pallas import tpu as pltpu
from jax.experimental.pallas import tpu_sc as plsc

_NC = 2
_NS = 16
_NW = _NC * _NS
_LANES = 16


def _project_table(embedding, W, b):
    V, E = embedding.shape
    D = W.shape[1]
    ET = embedding.T
    BLK = 8192
    grid = (V + BLK - 1) // BLK

    def mm(et_ref, w_ref, b_ref, o_ref):
        o_ref[...] = (
            jax.lax.dot_general(
                et_ref[...],
                w_ref[...],
                dimension_numbers=(((0,), (0,)), ((), ())),
                preferred_element_type=jnp.float32,
            )
            + b_ref[...]
        )

    return pl.pallas_call(
        mm,
        grid=(grid,),
        in_specs=[
            pl.BlockSpec((E, BLK), lambda i: (0, i)),
            pl.BlockSpec((E, D), lambda i: (0, 0)),
            pl.BlockSpec((1, D), lambda i: (0, 0)),
        ],
        out_specs=pl.BlockSpec((BLK, D), lambda i: (i, 0)),
        out_shape=jax.ShapeDtypeStruct((V, D), jnp.float32),
    )(ET, W, b.reshape(1, D))


def _rsqrt16(t):
    i = lax.bitcast_convert_type(t, jnp.int32)
    i = jnp.int32(0x5F3759DF) - lax.shift_right_arithmetic(i, 1)
    y = lax.bitcast_convert_type(i, jnp.float32)
    half_t = 0.5 * t
    for _ in range(3):
        y = y * (1.5 - half_t * y * y)
    return y


def _gather_normalize(P, x_flat, xt_flat, B, L, D):
    N = x_flat.shape[0]
    PER_W = N // _NW
    BPW = B // _NW
    BG = 8
    GR = BG * L
    NG = BPW // BG
    NCH = D // _LANES
    SPLIT = 128
    NSP = (GR + SPLIT - 1) // SPLIT
    assert BPW % BG == 0 and GR % 8 == 0 and NG % 2 == 0 and BPW == SPLIT

    mesh = plsc.VectorSubcoreMesh(core_axis_name="c", subcore_axis_name="s")

    @functools.partial(
        pl.kernel,
        mesh=mesh,
        out_type=jax.ShapeDtypeStruct((N, D), jnp.float32),
        scratch_types=[
            pltpu.VMEM((GR, D), jnp.float32),
            pltpu.VMEM((GR, D), jnp.float32),
            pltpu.VMEM((BPW, D), jnp.float32),
            pltpu.VMEM((L, SPLIT), jnp.int32),
            pltpu.VMEM((GR,), jnp.int32),
            pltpu.VMEM((GR,), jnp.int32),
            pltpu.SemaphoreType.DMA,
            pltpu.SemaphoreType.DMA,
            pltpu.SemaphoreType.DMA,
            pltpu.SemaphoreType.DMA,
            pltpu.SemaphoreType.DMA,
            pltpu.SemaphoreType.DMA,
        ],
    )
    def sc_kernel(
        p_hbm, xf_hbm, xtf_hbm, o_hbm,
        buf0, buf1, scales_v, idxt, ig0, ig1,
        g0, g1, s0, s1, isem, itsem,
    ):
        wid = lax.axis_index("s") * _NC + lax.axis_index("c")
        base = wid * PER_W
        wb = wid * BPW
        bufs = (buf0, buf1)
        igs = (ig0, ig1)
        gsems = (g0, g1)
        ssems = (s0, s1)

        def idx1_copy(g, k):
            return pltpu.make_async_copy(
                xf_hbm.at[pl.ds(base + g * GR, GR)], igs[k], isem
            )

        def idxt_copy(l):
            return pltpu.make_async_copy(
                xtf_hbm.at[pl.ds(l * B + wb, SPLIT)], idxt.at[l], itsem
            )

        def p1_gather_copies(k):
            out = []
            for s in range(NSP):
                ln = min(SPLIT, GR - s * SPLIT)
                out.append(
                    pltpu.make_async_copy(
                        p_hbm.at[igs[k].at[pl.ds(s * SPLIT, ln)]],
                        bufs[k].at[pl.ds(s * SPLIT, ln)],
                        gsems[k],
                    )
                )
            return out

        def p1_issue(k):
            for c in p1_gather_copies(k):
                c.start()

        def p1_wait(k):
            for c in p1_gather_copies(k):
                c.wait()

        def p1_compute(g, k):
            ib = bufs[k]
            for r in range(BG):
                rb = r * L

                def acc_body(l5, accs):
                    new = list(accs)
                    for u in range(5):
                        row = rb + 5 * l5 + u
                        for c in range(NCH):
                            v = ib[row, pl.ds(_LANES * c, _LANES)]
                            new[c] = new[c] + v * v
                    return tuple(new)

                zeros = tuple(jnp.zeros((_LANES,), jnp.float32) for _ in range(NCH))
                accs = lax.fori_loop(0, L // 5, acc_body, zeros)
                brow = g * BG + r
                for c in range(NCH):
                    scales_v[brow, pl.ds(_LANES * c, _LANES)] = _rsqrt16(
                        jnp.maximum(accs[c], jnp.float32(1e-24))
                    )

        def p2_gather_copy(l, k):
            return pltpu.make_async_copy(
                p_hbm.at[idxt.at[l]], bufs[k].at[pl.ds(0, SPLIT)], gsems[k]
            )

        def p2_store_copy(l, k):
            return pltpu.make_async_copy(
                bufs[k].at[pl.ds(SPLIT, SPLIT)],
                o_hbm.at[pl.ds(l * B + wb, SPLIT)],
                ssems[k],
            )

        def p2_compute(k):
            bf = bufs[k]

            def row_body(r4, carry):
                for u in range(4):
                    r = 4 * r4 + u
                    for c in range(NCH):
                        sl = pl.ds(_LANES * c, _LANES)
                        v = bf[r, sl] * scales_v[r, sl]
                        bf[SPLIT + r, sl] = jnp.maximum(v, 0.0)
                return carry

            lax.fori_loop(0, SPLIT // 4, row_body, 0)

        for l in range(L):
            idxt_copy(l).start()
        idx1_copy(0, 0).start()
        idx1_copy(0, 0).wait()
        p1_issue(0)
        idx1_copy(1, 1).start()

        def p1_outer(gp, carry):
            for k in range(2):
                g = 2 * gp + k
                p1_wait(k)

                @pl.when(g + 1 < NG)
                def _():
                    idx1_copy(g + 1, 1 - k).wait()
                    p1_issue(1 - k)

                @pl.when(g + 2 < NG)
                def _():
                    idx1_copy(g + 2, k).start()

                p1_compute(g, k)
            return carry

        lax.fori_loop(0, NG // 2, p1_outer, 0)

        for l in range(L):
            idxt_copy(l).wait()

        p2_gather_copy(0, 0).start()
        p2_gather_copy(1, 1).start()

        def p2_outer(lp, carry):
            for k in range(2):
                l = 2 * lp + k
                p2_gather_copy(l, k).wait()

                @pl.when(lp >= 1)
                def _():
                    p2_store_copy(l - 2, k).wait()

                p2_compute(k)
                p2_store_copy(l, k).start()

                @pl.when(lp < L // 2 - 1)
                def _():
                    p2_gather_copy(l + 2, k).start()
            return carry

        lax.fori_loop(0, L // 2, p2_outer, 0)
        p2_store_copy(L - 2, 0).wait()
        p2_store_copy(L - 1, 1).wait()

    return sc_kernel(P, x_flat, xt_flat)


def kernel(x, embedding, W, b):
    B, L = x.shape
    D = W.shape[1]
    P = _project_table(embedding, W, b)
    x_flat = x.reshape(B * L).astype(jnp.int32)
    xt_flat = x.T.reshape(L * B).astype(jnp.int32)
    out_t = _gather_normalize(P, x_flat, xt_flat, B, L, D)
    return out_t.reshape(L, B, D).transpose(1, 0, 2)


# speedup vs baseline: 1.0056x; 1.0056x over previous
import functools

import jax
import jax.numpy as jnp
from jax import lax
from jax.experimental import pallas as pl
from jax.experimental.pallas import tpu as pltpu
from jax.experimental.pallas import tpu_sc as plsc

_NC = 2
_NS = 16
_NW = _NC * _NS
_LANES = 16


def _project_table(embedding, W, b):
    V, E = embedding.shape
    D = W.shape[1]
    ET = embedding.T
    BLK = 8192
    grid = (V + BLK - 1) // BLK

    def mm(et_ref, w_ref, b_ref, o_ref):
        o_ref[...] = (
            jax.lax.dot_general(
                et_ref[...],
                w_ref[...],
                dimension_numbers=(((0,), (0,)), ((), ())),
                preferred_element_type=jnp.float32,
            )
            + b_ref[...]
        )

    return pl.pallas_call(
        mm,
        grid=(grid,),
        in_specs=[
            pl.BlockSpec((E, BLK), lambda i: (0, i)),
            pl.BlockSpec((E, D), lambda i: (0, 0)),
            pl.BlockSpec((1, D), lambda i: (0, 0)),
        ],
        out_specs=pl.BlockSpec((BLK, D), lambda i: (i, 0)),
        out_shape=jax.ShapeDtypeStruct((V, D), jnp.float32),
    )(ET, W, b.reshape(1, D))


def _rsqrt16(t):
    i = lax.bitcast_convert_type(t, jnp.int32)
    i = jnp.int32(0x5F3759DF) - lax.shift_right_arithmetic(i, 1)
    y = lax.bitcast_convert_type(i, jnp.float32)
    half_t = 0.5 * t
    for _ in range(3):
        y = y * (1.5 - half_t * y * y)
    return y


def _gather_normalize(P, x_flat, xt_flat, B, L, D):
    N = x_flat.shape[0]
    PER_W = N // _NW
    BPW = B // _NW
    BG = 8
    GR = BG * L
    NG = BPW // BG
    NCH = D // _LANES
    SPLIT = 128
    NSP = (GR + SPLIT - 1) // SPLIT
    assert BPW % BG == 0 and GR % 8 == 0 and NG % 2 == 0 and BPW == SPLIT

    mesh = plsc.VectorSubcoreMesh(core_axis_name="c", subcore_axis_name="s")

    @functools.partial(
        pl.kernel,
        mesh=mesh,
        out_type=jax.ShapeDtypeStruct((N, D), jnp.float32),
        scratch_types=[
            pltpu.VMEM((GR, D), jnp.float32),
            pltpu.VMEM((GR, D), jnp.float32),
            pltpu.VMEM((BPW, D), jnp.float32),
            pltpu.VMEM((L, SPLIT), jnp.int32),
            pltpu.VMEM((GR,), jnp.int32),
            pltpu.VMEM((GR,), jnp.int32),
            pltpu.SemaphoreType.DMA,
            pltpu.SemaphoreType.DMA,
            pltpu.SemaphoreType.DMA,
            pltpu.SemaphoreType.DMA,
            pltpu.SemaphoreType.DMA,
        ],
    )
    def sc_kernel(
        p_hbm, xf_hbm, xtf_hbm, o_hbm,
        buf0, buf1, scales_v, idxt, ig0, ig1,
        g0, g1, s0, s1, isem,
    ):
        wid = lax.axis_index("s") * _NC + lax.axis_index("c")
        base = wid * PER_W
        wb = wid * BPW
        bufs = (buf0, buf1)
        igs = (ig0, ig1)
        gsems = (g0, g1)
        ssems = (s0, s1)

        def idx1_copy(g, k):
            return pltpu.make_async_copy(
                xf_hbm.at[pl.ds(base + g * GR, GR)], igs[k], isem
            )

        def idxt_copy(l):
            return pltpu.make_async_copy(
                xtf_hbm.at[pl.ds(l * B + wb, SPLIT)], idxt.at[l], isem
            )

        def p1_gather_copies(k):
            out = []
            for s in range(NSP):
                ln = min(SPLIT, GR - s * SPLIT)
                out.append(
                    pltpu.make_async_copy(
                        p_hbm.at[igs[k].at[pl.ds(s * SPLIT, ln)]],
                        bufs[k].at[pl.ds(s * SPLIT, ln)],
                        gsems[k],
                    )
                )
            return out

        def p1_issue(k):
            for c in p1_gather_copies(k):
                c.start()

        def p1_wait(k):
            for c in p1_gather_copies(k):
                c.wait()

        def p1_compute(g, k):
            ib = bufs[k]
            for r in range(BG):
                rb = r * L

                def acc_body(l, accs):
                    row = rb + l
                    new = []
                    for c in range(NCH):
                        v = ib[row, pl.ds(_LANES * c, _LANES)]
                        new.append(accs[c] + v * v)
                    return tuple(new)

                zeros = tuple(jnp.zeros((_LANES,), jnp.float32) for _ in range(NCH))
                accs = lax.fori_loop(0, L, acc_body, zeros)
                brow = g * BG + r
                for c in range(NCH):
                    scales_v[brow, pl.ds(_LANES * c, _LANES)] = _rsqrt16(
                        jnp.maximum(accs[c], jnp.float32(1e-24))
                    )

        def p2_gather_copy(l, k):
            return pltpu.make_async_copy(
                p_hbm.at[idxt.at[l]], bufs[k].at[pl.ds(0, SPLIT)], gsems[k]
            )

        def p2_store_copy(l, k):
            return pltpu.make_async_copy(
                bufs[k].at[pl.ds(SPLIT, SPLIT)],
                o_hbm.at[pl.ds(l * B + wb, SPLIT)],
                ssems[k],
            )

        def p2_compute(k):
            bf = bufs[k]

            def row_body(r, carry):
                for c in range(NCH):
                    sl = pl.ds(_LANES * c, _LANES)
                    v = bf[r, sl] * scales_v[r, sl]
                    bf[SPLIT + r, sl] = jnp.maximum(v, 0.0)
                return carry

            lax.fori_loop(0, SPLIT, row_body, 0)

        idx1_copy(0, 0).start()
        idx1_copy(0, 0).wait()
        p1_issue(0)
        idx1_copy(1, 1).start()

        def p1_outer(gp, carry):
            for k in range(2):
                g = 2 * gp + k
                p1_wait(k)

                @pl.when(g + 1 < NG)
                def _():
                    idx1_copy(g + 1, 1 - k).wait()
                    p1_issue(1 - k)

                @pl.when(g + 2 < NG)
                def _():
                    idx1_copy(g + 2, k).start()

                p1_compute(g, k)
            return carry

        lax.fori_loop(0, NG // 2, p1_outer, 0)

        for l in range(L):
            idxt_copy(l).start()
        for l in range(L):
            idxt_copy(l).wait()

        p2_gather_copy(0, 0).start()
        p2_gather_copy(1, 1).start()

        def p2_outer(lp, carry):
            for k in range(2):
                l = 2 * lp + k
                p2_gather_copy(l, k).wait()

                @pl.when(lp >= 1)
                def _():
                    p2_store_copy(l - 2, k).wait()

                p2_compute(k)
                p2_store_copy(l, k).start()

                @pl.when(lp < L // 2 - 1)
                def _():
                    p2_gather_copy(l + 2, k).start()
            return carry

        lax.fori_loop(0, L // 2, p2_outer, 0)
        p2_store_copy(L - 2, 0).wait()
        p2_store_copy(L - 1, 1).wait()

    return sc_kernel(P, x_flat, xt_flat)


def kernel(x, embedding, W, b):
    B, L = x.shape
    D = W.shape[1]
    P = _project_table(embedding, W, b)
    x_flat = x.reshape(B * L).astype(jnp.int32)
    xt_flat = x.T.reshape(L * B).astype(jnp.int32)
    out_t = _gather_normalize(P, x_flat, xt_flat, B, L, D)
    return out_t.reshape(L, B, D).transpose(1, 0, 2)
